# baseline (speedup 1.0000x reference)
import jax
import jax.numpy as jnp
from jax import lax
from jax.experimental import pallas as pl
from jax.experimental.pallas import tpu as pltpu
from jax.experimental.pallas import tpu_sc as plsc

_B = 16
_SEQ = 2048
_FULL = 4096
_D = 512
_TBL = _SEQ + 2
_OUTL = 1 + _FULL
_NC = 2
_NS = 16
_NW = _NC * _NS
_ROWS_PER_W = (_B * _FULL) // _NW
_CHUNK = 128
_NCHUNK = _ROWS_PER_W // _CHUNK


def _iota16():
    return lax.iota(jnp.int32, 16)


def _body(table_hbm, idx_hbm, out_hbm, idx_v, src_v, dst_v, buf,
          g_src_v, g_dst_v, g_buf, sem):
    c = lax.axis_index("c")
    s = lax.axis_index("s")
    wid = s * _NC + c
    b = wid // 2
    half = wid % 2
    base_t = b * _FULL + half * _SEQ
    out_base = b * _OUTL + 1 + half * _SEQ
    tbl_base = b * _TBL

    pltpu.sync_copy(idx_hbm.at[pl.ds(base_t, _ROWS_PER_W)], idx_v)

    @pl.when(wid == 0)
    def _():
        g_src_v[...] = _iota16() * _TBL
        g_dst_v[...] = _iota16() * _OUTL
        pltpu.async_copy(table_hbm.at[g_src_v], g_buf, sem).wait()
        pltpu.async_copy(g_buf, out_hbm.at[g_dst_v], sem).wait()

    for g in range(_NCHUNK):
        for v in range(_CHUNK // 16):
            vec = idx_v[pl.ds(g * _CHUNK + v * 16, 16)]
            valid = vec < _SEQ
            src = jnp.where(valid, vec + (tbl_base + 1),
                            jnp.full((16,), 0, jnp.int32) + (tbl_base + _TBL - 1))
            src_v[pl.ds(v * 16, 16)] = src
            dst_v[pl.ds(v * 16, 16)] = _iota16() + (out_base + g * _CHUNK + v * 16)
        pltpu.async_copy(table_hbm.at[src_v], buf, sem).wait()
        pltpu.async_copy(buf, out_hbm.at[dst_v], sem).wait()


@jax.jit
def _revert(table, idx):
    mesh = plsc.VectorSubcoreMesh(core_axis_name="c", subcore_axis_name="s",
                                  num_cores=_NC, num_subcores=_NS)
    return pl.kernel(
        _body,
        out_type=jax.ShapeDtypeStruct((_B * _OUTL, _D), jnp.float32),
        mesh=mesh,
        scratch_types=[
            pltpu.VMEM((_ROWS_PER_W,), jnp.int32),
            pltpu.VMEM((_CHUNK,), jnp.int32),
            pltpu.VMEM((_CHUNK,), jnp.int32),
            pltpu.VMEM((_CHUNK, _D), jnp.float32),
            pltpu.VMEM((16,), jnp.int32),
            pltpu.VMEM((16,), jnp.int32),
            pltpu.VMEM((16, _D), jnp.float32),
            pltpu.SemaphoreType.DMA,
        ],
    )(table, idx)


def kernel(data, revert_idx, mask_token):
    table = jnp.concatenate(
        [data, jnp.broadcast_to(mask_token[None, None, :], (_B, 1, _D))], axis=1)
    table = table.reshape(_B * _TBL, _D)
    idx = revert_idx.reshape(_B * _FULL)
    out = _revert(table, idx)
    return out.reshape(_B, _OUTL, _D)


# speedup vs baseline: 1.0410x; 1.0410x over previous
import jax
import jax.numpy as jnp
from jax import lax
from jax.experimental import pallas as pl
from jax.experimental.pallas import tpu as pltpu
from jax.experimental.pallas import tpu_sc as plsc

_B = 16
_SEQ = 2048
_FULL = 4096
_D = 512
_DL = _SEQ + 1
_OUTL = 1 + _FULL
_NC = 2
_NS = 16
_NW = _NC * _NS
_RPW = (_B * _FULL) // _NW
_CH = 64
_NPAIR = _RPW // (2 * _CH)


def _iota16():
    return lax.iota(jnp.int32, 16)


def _zeros16():
    return jnp.zeros((16,), jnp.int32)


def _body(data_hbm, idx_hbm, mask_hbm, out_hbm,
          idx_v, sr0, sr1, dd0, dd1, buf0, buf1, mask_v,
          g_src, g_dst, g_buf, sem, g0, g1, s0, s1):
    c = lax.axis_index("c")
    s = lax.axis_index("s")
    wid = s * _NC + c
    b = wid // 2
    half = wid % 2
    base_t = b * _FULL + half * _SEQ
    out_base = b * _OUTL + 1 + half * _SEQ
    dbase = b * _DL

    pltpu.sync_copy(idx_hbm.at[pl.ds(base_t, _RPW)], idx_v)
    pltpu.sync_copy(mask_hbm, mask_v)

    @pl.when(wid == 0)
    def _():
        g_src[...] = _iota16() * _DL
        g_dst[...] = _iota16() * _OUTL
        pltpu.async_copy(data_hbm.at[g_src], g_buf, sem).wait()
        pltpu.async_copy(g_buf, out_hbm.at[g_dst], sem).wait()

    mvec = [mask_v[0, pl.ds(cc * 16, 16)] for cc in range(_D // 16)]

    def build(j, sr, dd):
        for v in range(_CH // 16):
            vec = idx_v[pl.ds(j * _CH + v * 16, 16)]
            valid = vec < _SEQ
            sr[pl.ds(v * 16, 16)] = jnp.where(valid, vec + (dbase + 1),
                                              _zeros16() + dbase)
            dd[pl.ds(v * 16, 16)] = _iota16() + (out_base + j * _CH + v * 16)

    def patch(j, buf):
        for r in range(_CH):
            g, l = divmod(r, 16)
            vv = idx_v[pl.ds(j * _CH + g * 16, 16)]
            val = vv[l]

            @pl.when(val >= _SEQ)
            def _():
                for cc in range(_D // 16):
                    buf[r, pl.ds(cc * 16, 16)] = mvec[cc]

    def pair(p, _):
        ja = 2 * p
        jb = 2 * p + 1

        @pl.when(p > 0)
        def _():
            pltpu.make_async_copy(buf0, out_hbm.at[dd0], s0).wait()
            pltpu.make_async_copy(buf1, out_hbm.at[dd1], s1).wait()

        build(ja, sr0, dd0)
        build(jb, sr1, dd1)

        ga = pltpu.async_copy(data_hbm.at[sr0], buf0, g0)
        gb = pltpu.async_copy(data_hbm.at[sr1], buf1, g1)
        ga.wait()
        patch(ja, buf0)
        pltpu.async_copy(buf0, out_hbm.at[dd0], s0)
        gb.wait()
        patch(jb, buf1)
        pltpu.async_copy(buf1, out_hbm.at[dd1], s1)
        return 0

    lax.fori_loop(0, _NPAIR, pair, 0)
    pltpu.make_async_copy(buf0, out_hbm.at[dd0], s0).wait()
    pltpu.make_async_copy(buf1, out_hbm.at[dd1], s1).wait()


@jax.jit
def _revert(data_flat, idx, mask_row):
    mesh = plsc.VectorSubcoreMesh(core_axis_name="c", subcore_axis_name="s",
                                  num_cores=_NC, num_subcores=_NS)
    return pl.kernel(
        _body,
        out_type=jax.ShapeDtypeStruct((_B * _OUTL, _D), jnp.float32),
        mesh=mesh,
        scratch_types=[
            pltpu.VMEM((_RPW,), jnp.int32),
            pltpu.VMEM((_CH,), jnp.int32),
            pltpu.VMEM((_CH,), jnp.int32),
            pltpu.VMEM((_CH,), jnp.int32),
            pltpu.VMEM((_CH,), jnp.int32),
            pltpu.VMEM((_CH, _D), jnp.float32),
            pltpu.VMEM((_CH, _D), jnp.float32),
            pltpu.VMEM((1, _D), jnp.float32),
            pltpu.VMEM((16,), jnp.int32),
            pltpu.VMEM((16,), jnp.int32),
            pltpu.VMEM((16, _D), jnp.float32),
            pltpu.SemaphoreType.DMA,
            pltpu.SemaphoreType.DMA,
            pltpu.SemaphoreType.DMA,
            pltpu.SemaphoreType.DMA,
            pltpu.SemaphoreType.DMA,
        ],
    )(data_flat, idx, mask_row)


def kernel(data, revert_idx, mask_token):
    data_flat = data.reshape(_B * _DL, _D)
    idx = revert_idx.reshape(_B * _FULL)
    mask_row = mask_token.reshape(1, _D)
    out = _revert(data_flat, idx, mask_row)
    return out.reshape(_B, _OUTL, _D)


# speedup vs baseline: 1.2555x; 1.2060x over previous
import jax
import jax.numpy as jnp
from jax import lax
from jax.experimental import pallas as pl
from jax.experimental.pallas import tpu as pltpu
from jax.experimental.pallas import tpu_sc as plsc

_B = 16
_SEQ = 2048
_FULL = 4096
_D = 512
_DL = _SEQ + 1
_OUTL = 1 + _FULL
_NC = 2
_NS = 16
_NW = _NC * _NS
_RPW = (_B * _FULL) // _NW
_CH = 64
_NPAIR = _RPW // (2 * _CH)


def _iota16():
    return lax.iota(jnp.int32, 16)


def _zeros16():
    return jnp.zeros((16,), jnp.int32)


def _body(data_hbm, idx_hbm, mask_hbm, out_hbm,
          idx_v, sr0, sr1, dd0, dd1, buf0, buf1, mask_v,
          g_src, g_buf, sem, g0, g1, s0, s1):
    c = lax.axis_index("c")
    s = lax.axis_index("s")
    wid = s * _NC + c
    b = wid // 2
    half = wid % 2
    base_t = b * _FULL + half * _SEQ
    out_base = 1 + half * _SEQ

    dview = data_hbm.at[b]
    oview = out_hbm.at[b]

    pltpu.sync_copy(idx_hbm.at[pl.ds(base_t, _RPW)], idx_v)
    pltpu.sync_copy(mask_hbm, mask_v)

    @pl.when(half == 0)
    def _():
        g_src[...] = _zeros16()
        pltpu.async_copy(dview.at[g_src], g_buf, sem).wait()
        pltpu.sync_copy(g_buf.at[pl.ds(0, 1)], oview.at[pl.ds(0, 1)])

    mvec = [mask_v[0, pl.ds(cc * 16, 16)] for cc in range(_D // 16)]

    def build(j, sr, dd):
        for v in range(_CH // 16):
            vec = idx_v[pl.ds(j * _CH + v * 16, 16)]
            valid = vec < _SEQ
            sr[pl.ds(v * 16, 16)] = jnp.where(valid, vec + 1, _zeros16())
            dd[pl.ds(v * 16, 16)] = _iota16() + (out_base + j * _CH + v * 16)

    def patch(j, buf):
        for r in range(_CH):
            g, l = divmod(r, 16)
            vv = idx_v[pl.ds(j * _CH + g * 16, 16)]
            val = vv[l]

            @pl.when(val >= _SEQ)
            def _():
                for cc in range(_D // 16):
                    buf[r, pl.ds(cc * 16, 16)] = mvec[cc]

    def pair(p, _):
        ja = 2 * p
        jb = 2 * p + 1

        @pl.when(p > 0)
        def _():
            pltpu.make_async_copy(buf0, oview.at[dd0], s0).wait()
            pltpu.make_async_copy(buf1, oview.at[dd1], s1).wait()

        build(ja, sr0, dd0)
        build(jb, sr1, dd1)
        ga = pltpu.async_copy(dview.at[sr0], buf0, g0)
        gb = pltpu.async_copy(dview.at[sr1], buf1, g1)
        ga.wait()
        patch(ja, buf0)
        pltpu.async_copy(buf0, oview.at[dd0], s0)
        gb.wait()
        patch(jb, buf1)
        pltpu.async_copy(buf1, oview.at[dd1], s1)
        return 0

    lax.fori_loop(0, _NPAIR, pair, 0)
    pltpu.make_async_copy(buf0, oview.at[dd0], s0).wait()
    pltpu.make_async_copy(buf1, oview.at[dd1], s1).wait()


@jax.jit
def _revert(data, idx, mask_row):
    mesh = plsc.VectorSubcoreMesh(core_axis_name="c", subcore_axis_name="s",
                                  num_cores=_NC, num_subcores=_NS)
    return pl.kernel(
        _body,
        out_type=jax.ShapeDtypeStruct((_B, _OUTL, _D), jnp.float32),
        mesh=mesh,
        scratch_types=[
            pltpu.VMEM((_RPW,), jnp.int32),
            pltpu.VMEM((_CH,), jnp.int32),
            pltpu.VMEM((_CH,), jnp.int32),
            pltpu.VMEM((_CH,), jnp.int32),
            pltpu.VMEM((_CH,), jnp.int32),
            pltpu.VMEM((_CH, _D), jnp.float32),
            pltpu.VMEM((_CH, _D), jnp.float32),
            pltpu.VMEM((1, _D), jnp.float32),
            pltpu.VMEM((16,), jnp.int32),
            pltpu.VMEM((16, _D), jnp.float32),
            pltpu.SemaphoreType.DMA,
            pltpu.SemaphoreType.DMA,
            pltpu.SemaphoreType.DMA,
            pltpu.SemaphoreType.DMA,
            pltpu.SemaphoreType.DMA,
        ],
    )(data, idx, mask_row)


def kernel(data, revert_idx, mask_token):
    idx = revert_idx.reshape(_B * _FULL)
    mask_row = mask_token.reshape(1, _D)
    return _revert(data, idx, mask_row)


# speedup vs baseline: 1.3452x; 1.0715x over previous
import jax
import jax.numpy as jnp
from jax import lax
from jax.experimental import pallas as pl
from jax.experimental.pallas import tpu as pltpu
from jax.experimental.pallas import tpu_sc as plsc

_B = 16
_SEQ = 2048
_FULL = 4096
_D = 512
_DL = _SEQ + 1
_OUTL = 1 + _FULL
_NC = 2
_NS = 16
_NW = _NC * _NS
_RPW = (_B * _FULL) // _NW
_CH = 64
_NCH = _RPW // _CH


def _iota16():
    return lax.iota(jnp.int32, 16)


def _zeros16():
    return jnp.zeros((16,), jnp.int32)


def _body(data_hbm, idx_hbm, mask_hbm, out_hbm,
          idx_v, sr0, sr1, dd0, dd1, buf0, buf1, mask_v,
          g_src, g_buf, sem, g0, g1, s0, s1):
    c = lax.axis_index("c")
    s = lax.axis_index("s")
    wid = s * _NC + c
    b = wid // 2
    half = wid % 2
    base_t = b * _FULL + half * _SEQ
    out_base = 1 + half * _SEQ

    dview = data_hbm.at[b]
    oview = out_hbm.at[b]

    pltpu.sync_copy(idx_hbm.at[pl.ds(base_t, _RPW)], idx_v)
    pltpu.sync_copy(mask_hbm, mask_v)

    @pl.when(half == 0)
    def _():
        pltpu.sync_copy(dview.at[pl.ds(0, 1)], oview.at[pl.ds(0, 1)])

    mvec = [mask_v[0, pl.ds(cc * 16, 16)] for cc in range(_D // 16)]

    def build(j, sr, dd):
        for v in range(_CH // 16):
            vec = idx_v[pl.ds(j * _CH + v * 16, 16)]
            valid = vec < _SEQ
            sr[pl.ds(v * 16, 16)] = jnp.where(valid, vec + 1, _zeros16())
            dd[pl.ds(v * 16, 16)] = _iota16() + (out_base + j * _CH + v * 16)

    def patch(j, buf):
        for r in range(_CH):
            g, l = divmod(r, 16)
            vv = idx_v[pl.ds(j * _CH + g * 16, 16)]
            val = vv[l]

            @pl.when(val >= _SEQ)
            def _():
                for cc in range(_D // 16):
                    buf[r, pl.ds(cc * 16, 16)] = mvec[cc]

    def halfstep(j, bufa, sra, dda, ga, sa, bufb, srb, ddb, gb, sb):
        pltpu.make_async_copy(dview.at[sra], bufa, ga).wait()

        @pl.when(j > 0)
        def _():
            pltpu.make_async_copy(bufb, oview.at[ddb], sb).wait()

        @pl.when(j < _NCH - 1)
        def _():
            build(j + 1, srb, ddb)
            pltpu.async_copy(dview.at[srb], bufb, gb)

        patch(j, bufa)
        pltpu.async_copy(bufa, oview.at[dda], sa)

    def step(j, _):
        @pl.when(j % 2 == 0)
        def _():
            halfstep(j, buf0, sr0, dd0, g0, s0, buf1, sr1, dd1, g1, s1)

        @pl.when(j % 2 == 1)
        def _():
            halfstep(j, buf1, sr1, dd1, g1, s1, buf0, sr0, dd0, g0, s0)

        return 0

    build(0, sr0, dd0)
    pltpu.async_copy(dview.at[sr0], buf0, g0)
    lax.fori_loop(0, _NCH, step, 0)
    pltpu.make_async_copy(buf1, oview.at[dd1], s1).wait()


@jax.jit
def _revert(data, idx, mask_row):
    mesh = plsc.VectorSubcoreMesh(core_axis_name="c", subcore_axis_name="s",
                                  num_cores=_NC, num_subcores=_NS)
    return pl.kernel(
        _body,
        out_type=jax.ShapeDtypeStruct((_B, _OUTL, _D), jnp.float32),
        mesh=mesh,
        scratch_types=[
            pltpu.VMEM((_RPW,), jnp.int32),
            pltpu.VMEM((_CH,), jnp.int32),
            pltpu.VMEM((_CH,), jnp.int32),
            pltpu.VMEM((_CH,), jnp.int32),
            pltpu.VMEM((_CH,), jnp.int32),
            pltpu.VMEM((_CH, _D), jnp.float32),
            pltpu.VMEM((_CH, _D), jnp.float32),
            pltpu.VMEM((1, _D), jnp.float32),
            pltpu.VMEM((16,), jnp.int32),
            pltpu.VMEM((16, _D), jnp.float32),
            pltpu.SemaphoreType.DMA,
            pltpu.SemaphoreType.DMA,
            pltpu.SemaphoreType.DMA,
            pltpu.SemaphoreType.DMA,
            pltpu.SemaphoreType.DMA,
        ],
    )(data, idx, mask_row)


def kernel(data, revert_idx, mask_token):
    idx = revert_idx.reshape(_B * _FULL)
    mask_row = mask_token.reshape(1, _D)
    return _revert(data, idx, mask_row)
